# baseline (speedup 1.0000x reference)
import jax
import jax.numpy as jnp
from jax import lax
from jax.experimental import pallas as pl
from jax.experimental.pallas import tpu as pltpu
from jax.experimental.pallas import tpu_sc as plsc

_N = 10000
_NP = 10240
_E = 320000
_D = 128
_NS = 16
_CH = 125
_ER = _E // _CH
_RPT = _ER // _NS
_KC = 16
_NBUF = 2
_NPT = _NP // _NS
_ZW = 64
_EPAD = _ER * 128
_EPT = _EPAD // _NS
_DC = 1024

_MESH = plsc.VectorSubcoreMesh(core_axis_name="c", subcore_axis_name="s",
                               num_cores=1)


def _make_scatter():
  outs = jax.ShapeDtypeStruct((_NP, _D), jnp.float32)
  scratch = [pltpu.VMEM((_KC, _CH), jnp.int32) for _ in range(4)]
  scratch += [pltpu.VMEM((_CH, _D), jnp.float32) for _ in range(_NBUF)]
  scratch += [pltpu.VMEM_SHARED((_NP, _D), jnp.float32)]
  scratch += [pltpu.SemaphoreType.DMA for _ in range(_NBUF + 4)]

  def body(z, srch, dsth, aggo, sv0, dv0, sv1, dv1, b0, b1, aggs,
           s0, s1, ss0, sd0, ss1, sd1):
    bufs = (b0, b1)
    sems = (s0, s1)
    isets = ((sv0, dv0, ss0, sd0), (sv1, dv1, ss1, sd1))
    sid = lax.axis_index("s")

    z16 = jnp.zeros((16,), jnp.float32)

    @pl.loop(0, _ZW)
    def _zero(i):
      for j in range(_D // 16):
        b0[i, pl.ds(j * 16, 16)] = z16

    zb = b0.at[pl.ds(0, _ZW)]
    row0 = sid * _NPT
    for t in range(_NPT // _ZW):
      pltpu.sync_copy(zb, aggs.at[pl.ds(row0 + t * _ZW, _ZW)])
    plsc.subcore_barrier()

    ibase = sid * _RPT
    pltpu.sync_copy(srch.at[pl.ds(ibase, _KC)], sv0)
    pltpu.sync_copy(dsth.at[pl.ds(ibase, _KC)], dv0)
    for b in range(_NBUF):
      pltpu.async_copy(z.at[sv0.at[b]], bufs[b], sems[b])

    def run_chunk(kk, cur, nxt, has_next):
      sv, dv, _, _ = cur
      svn, dvn, ssn, sdn = nxt

      @pl.when(has_next)
      def _pfidx():
        pltpu.async_copy(srch.at[pl.ds(ibase + kk + _KC, _KC)], svn, ssn)
        pltpu.async_copy(dsth.at[pl.ds(ibase + kk + _KC, _KC)], dvn, sdn)

      @pl.loop(0, _KC - _NBUF, step=_NBUF)
      def _main(jj):
        for b in range(_NBUF):
          j = jj + b
          pltpu.make_async_copy(z.at[sv.at[j]], bufs[b], sems[b]).wait()
          pltpu.sync_copy(bufs[b], aggs.at[dv.at[j]], add=True)
          pltpu.async_copy(z.at[sv.at[j + _NBUF]], bufs[b], sems[b])

      @pl.when(has_next)
      def _tail():
        pltpu.make_async_copy(
            srch.at[pl.ds(ibase + kk + _KC, _KC)], svn, ssn).wait()
        pltpu.make_async_copy(
            dsth.at[pl.ds(ibase + kk + _KC, _KC)], dvn, sdn).wait()
        for b in range(_NBUF):
          j = _KC - _NBUF + b
          pltpu.make_async_copy(z.at[sv.at[j]], bufs[b], sems[b]).wait()
          pltpu.sync_copy(bufs[b], aggs.at[dv.at[j]], add=True)
          pltpu.async_copy(z.at[svn.at[b]], bufs[b], sems[b])

      @pl.when(jnp.logical_not(has_next))
      def _drain():
        for b in range(_NBUF):
          j = _KC - _NBUF + b
          pltpu.make_async_copy(z.at[sv.at[j]], bufs[b], sems[b]).wait()
          pltpu.sync_copy(bufs[b], aggs.at[dv.at[j]], add=True)

    @pl.loop(0, _RPT, step=2 * _KC)
    def _outer(kk):
      run_chunk(kk, isets[0], isets[1], kk + _KC < _RPT + 1)
      run_chunk(kk + _KC, isets[1], isets[0], kk + 2 * _KC < _RPT)

    plsc.subcore_barrier()
    pltpu.sync_copy(aggs.at[pl.ds(row0, _NPT)],
                    aggo.at[pl.ds(row0, _NPT)])

  return pl.kernel(body, out_type=outs, mesh=_MESH,
                   scratch_types=tuple(scratch))


def _make_deghist():
  outs = jax.ShapeDtypeStruct((_NS * _NP,), jnp.float32)
  scratch = [
      pltpu.VMEM((_DC,), jnp.int32),
      pltpu.VMEM((_NP,), jnp.float32),
  ]

  def body(dstpf, histo, dvc, hist):
    sid = lax.axis_index("s")
    z16 = jnp.zeros((16,), jnp.float32)
    ones16 = jnp.ones((16,), jnp.float32)

    @pl.loop(0, _NP // 16)
    def _zeroh(i):
      hist[pl.ds(i * 16, 16)] = z16

    ebase = sid * _EPT

    @pl.loop(0, _EPT, step=_DC)
    def _chunk(cc):
      pltpu.sync_copy(dstpf.at[pl.ds(ebase + cc, _DC)], dvc)

      @pl.loop(0, _DC // 16)
      def _cnt(m):
        idxv = dvc[pl.ds(m * 16, 16)]
        plsc.addupdate_scatter(hist, [idxv], ones16)

    pltpu.sync_copy(hist, histo.at[pl.ds(sid * _NP, _NP)])

  cp = pltpu.CompilerParams(needs_layout_passes=False)
  return pl.kernel(body, out_type=outs, mesh=_MESH,
                   scratch_types=tuple(scratch), compiler_params=cp)


_BR = 1000
_G = _N // _BR
_F32 = dict(preferred_element_type=jnp.float32,
            precision=lax.Precision.HIGHEST)


def _degsum(histo):
  def body(h_ref, o_ref):
    o_ref[...] = jnp.sum(h_ref[...], axis=0, keepdims=True)

  return pl.pallas_call(
      body,
      grid=(1,),
      in_specs=[pl.BlockSpec((_NS, _NP), lambda i: (0, 0))],
      out_specs=pl.BlockSpec((1, _NP), lambda i: (0, 0)),
      out_shape=jax.ShapeDtypeStruct((1, _NP), jnp.float32),
  )(histo)


def _selfproj(h, Ws, b):
  Din, Dout = Ws.shape

  def body(h_ref, ws_ref, b_ref, o_ref):
    o_ref[...] = jnp.dot(h_ref[...], ws_ref[...], **_F32) + b_ref[...]

  return pl.pallas_call(
      body,
      grid=(_G,),
      in_specs=[
          pl.BlockSpec((_BR, Din), lambda i: (i, 0)),
          pl.BlockSpec((Din, Dout), lambda i: (0, 0)),
          pl.BlockSpec((1, Dout), lambda i: (0, 0)),
      ],
      out_specs=pl.BlockSpec((_BR, Dout), lambda i: (i, 0)),
      out_shape=jax.ShapeDtypeStruct((_N, Dout), jnp.float32),
  )(h, Ws, b)


def _combine(h, Ws, Wn, b, q, degc, act):
  Din, Dout = Ws.shape

  def body(h_ref, ws_ref, wn_ref, b_ref, q_ref, d_ref, o_ref):
    invd = 1.0 / jnp.maximum(d_ref[...], 1.0)
    t = jnp.dot(h_ref[...], ws_ref[...], **_F32) + b_ref[...]
    t = t + jnp.dot(q_ref[...] * invd, wn_ref[...], **_F32)
    if act:
      t = jnp.maximum(t, 0.0)
    o_ref[...] = t

  return pl.pallas_call(
      body,
      grid=(_G,),
      in_specs=[
          pl.BlockSpec((_BR, Din), lambda i: (i, 0)),
          pl.BlockSpec((Din, Dout), lambda i: (0, 0)),
          pl.BlockSpec((Din, Dout), lambda i: (0, 0)),
          pl.BlockSpec((1, Dout), lambda i: (0, 0)),
          pl.BlockSpec((_BR, _D), lambda i: (i, 0)),
          pl.BlockSpec((_BR, 1), lambda i: (i, 0)),
      ],
      out_specs=pl.BlockSpec((_BR, Dout), lambda i: (i, 0)),
      out_shape=jax.ShapeDtypeStruct((_N, Dout), jnp.float32),
  )(h, Ws, Wn, b, q, degc)


def kernel(edge_index, inputs, W0s, W0n, b0, W1s, W1n, b1, W2s, W2n, b2):
  ei = edge_index.astype(jnp.int32)
  src = ei[0].reshape(_ER, _CH)
  dst = ei[1].reshape(_ER, _CH)
  dstpf = jnp.pad(dst, ((0, 0), (0, 128 - _CH)),
                  constant_values=_N + 100).reshape(-1)

  b0r = b0.reshape(1, -1)
  b1r = b1.reshape(1, -1)
  b2r = b2.reshape(1, -1)

  scat = _make_scatter()
  deghist = _make_deghist()

  histo = deghist(dstpf)
  degc = _degsum(histo.reshape(_NS, _NP)).reshape(_NP, 1)
  q = scat(inputs, src, dst)
  h1 = _combine(inputs, W0s, W0n, b0r, q, degc, False)
  q = scat(h1, src, dst)
  h2 = _combine(h1, W1s, W1n, b1r, q, degc, True)
  q = scat(h2, src, dst)
  h3 = _combine(h2, W1s, W1n, b1r, q, degc, True)
  q = scat(h3, src, dst)
  out = _combine(h3, W2s, W2n, b2r, q, degc, False)
  return out, h3


# speedup vs baseline: 1.0171x; 1.0171x over previous
import jax
import jax.numpy as jnp
from jax import lax
from jax.experimental import pallas as pl
from jax.experimental.pallas import tpu as pltpu
from jax.experimental.pallas import tpu_sc as plsc

_N = 10000
_NP = 10240
_E = 320000
_D = 128
_NS = 16
_CH = 125
_ER = _E // _CH
_RPT = _ER // _NS
_KC = 16
_NBUF = 2
_NPT = _NP // _NS
_ZW = 64
_EPAD = _ER * 128
_EPT = _EPAD // _NS
_DC = 1024

_MESH = plsc.VectorSubcoreMesh(core_axis_name="c", subcore_axis_name="s",
                               num_cores=1)


def _make_scatter():
  outs = jax.ShapeDtypeStruct((_NP, _D), jnp.float32)
  scratch = [pltpu.VMEM((_KC, _CH), jnp.int32) for _ in range(4)]
  scratch += [pltpu.VMEM((_CH, _D), jnp.float32) for _ in range(_NBUF)]
  scratch += [pltpu.VMEM_SHARED((_NP, _D), jnp.float32)]
  scratch += [pltpu.SemaphoreType.DMA for _ in range(_NBUF + 4)]

  def body(z, srch, dsth, aggo, sv0, dv0, sv1, dv1, b0, b1, aggs,
           s0, s1, ss0, sd0, ss1, sd1):
    bufs = (b0, b1)
    sems = (s0, s1)
    isets = ((sv0, dv0, ss0, sd0), (sv1, dv1, ss1, sd1))
    sid = lax.axis_index("s")

    z16 = jnp.zeros((16,), jnp.float32)

    @pl.loop(0, _ZW)
    def _zero(i):
      for j in range(_D // 16):
        b0[i, pl.ds(j * 16, 16)] = z16

    zb = b0.at[pl.ds(0, _ZW)]
    row0 = sid * _NPT
    for t in range(_NPT // _ZW):
      pltpu.sync_copy(zb, aggs.at[pl.ds(row0 + t * _ZW, _ZW)])
    plsc.subcore_barrier()

    ibase = sid * _RPT
    pltpu.sync_copy(srch.at[pl.ds(ibase, _KC)], sv0)
    pltpu.sync_copy(dsth.at[pl.ds(ibase, _KC)], dv0)
    for b in range(_NBUF):
      pltpu.async_copy(z.at[sv0.at[b]], bufs[b], sems[b])

    def run_chunk(kk, cur, nxt, has_next):
      sv, dv, _, _ = cur
      svn, dvn, ssn, sdn = nxt

      @pl.when(has_next)
      def _pfidx():
        pltpu.async_copy(srch.at[pl.ds(ibase + kk + _KC, _KC)], svn, ssn)
        pltpu.async_copy(dsth.at[pl.ds(ibase + kk + _KC, _KC)], dvn, sdn)

      @pl.loop(0, _KC - _NBUF, step=_NBUF)
      def _main(jj):
        for b in range(_NBUF):
          j = jj + b
          pltpu.make_async_copy(z.at[sv.at[j]], bufs[b], sems[b]).wait()
          pltpu.sync_copy(bufs[b], aggs.at[dv.at[j]], add=True)
          pltpu.async_copy(z.at[sv.at[j + _NBUF]], bufs[b], sems[b])

      @pl.when(has_next)
      def _tail():
        pltpu.make_async_copy(
            srch.at[pl.ds(ibase + kk + _KC, _KC)], svn, ssn).wait()
        pltpu.make_async_copy(
            dsth.at[pl.ds(ibase + kk + _KC, _KC)], dvn, sdn).wait()
        for b in range(_NBUF):
          j = _KC - _NBUF + b
          pltpu.make_async_copy(z.at[sv.at[j]], bufs[b], sems[b]).wait()
          pltpu.sync_copy(bufs[b], aggs.at[dv.at[j]], add=True)
          pltpu.async_copy(z.at[svn.at[b]], bufs[b], sems[b])

      @pl.when(jnp.logical_not(has_next))
      def _drain():
        for b in range(_NBUF):
          j = _KC - _NBUF + b
          pltpu.make_async_copy(z.at[sv.at[j]], bufs[b], sems[b]).wait()
          pltpu.sync_copy(bufs[b], aggs.at[dv.at[j]], add=True)

    @pl.loop(0, _RPT, step=2 * _KC)
    def _outer(kk):
      run_chunk(kk, isets[0], isets[1], kk + _KC < _RPT + 1)
      run_chunk(kk + _KC, isets[1], isets[0], kk + 2 * _KC < _RPT)

    plsc.subcore_barrier()
    pltpu.sync_copy(aggs.at[pl.ds(row0, _NPT)],
                    aggo.at[pl.ds(row0, _NPT)])

  return pl.kernel(body, out_type=outs, mesh=_MESH,
                   scratch_types=tuple(scratch))


def _make_deghist():
  outs = jax.ShapeDtypeStruct((_NS * _NP,), jnp.float32)
  scratch = [
      pltpu.VMEM((_DC,), jnp.int32),
      pltpu.VMEM((_NP,), jnp.float32),
  ]

  def body(dstpf, histo, dvc, hist):
    sid = lax.axis_index("s")
    z16 = jnp.zeros((16,), jnp.float32)
    ones16 = jnp.ones((16,), jnp.float32)

    @pl.loop(0, _NP // 16)
    def _zeroh(i):
      hist[pl.ds(i * 16, 16)] = z16

    ebase = sid * _EPT

    @pl.loop(0, _EPT, step=_DC)
    def _chunk(cc):
      pltpu.sync_copy(dstpf.at[pl.ds(ebase + cc, _DC)], dvc)

      @pl.loop(0, _DC // 16)
      def _cnt(m):
        idxv = dvc[pl.ds(m * 16, 16)]
        plsc.addupdate_scatter(hist, [idxv], ones16)

    pltpu.sync_copy(hist, histo.at[pl.ds(sid * _NP, _NP)])

  cp = pltpu.CompilerParams(needs_layout_passes=False)
  return pl.kernel(body, out_type=outs, mesh=_MESH,
                   scratch_types=tuple(scratch), compiler_params=cp)


_BR = 1000
_G = _N // _BR
_F32 = dict(preferred_element_type=jnp.float32,
            precision=lax.Precision.HIGHEST)


def _degsum(histo):
  def body(h_ref, o_ref):
    o_ref[...] = jnp.sum(h_ref[...], axis=0, keepdims=True)

  return pl.pallas_call(
      body,
      grid=(1,),
      in_specs=[pl.BlockSpec((_NS, _NP), lambda i: (0, 0))],
      out_specs=pl.BlockSpec((1, _NP), lambda i: (0, 0)),
      out_shape=jax.ShapeDtypeStruct((1, _NP), jnp.float32),
  )(histo)


def _selfproj(h, Ws, b):
  Din, Dout = Ws.shape

  def body(h_ref, ws_ref, b_ref, o_ref):
    o_ref[...] = jnp.dot(h_ref[...], ws_ref[...], **_F32) + b_ref[...]

  return pl.pallas_call(
      body,
      grid=(_G,),
      in_specs=[
          pl.BlockSpec((_BR, Din), lambda i: (i, 0)),
          pl.BlockSpec((Din, Dout), lambda i: (0, 0)),
          pl.BlockSpec((1, Dout), lambda i: (0, 0)),
      ],
      out_specs=pl.BlockSpec((_BR, Dout), lambda i: (i, 0)),
      out_shape=jax.ShapeDtypeStruct((_N, Dout), jnp.float32),
  )(h, Ws, b)


def _combine(s, Wn, q, degc, act):
  Dout = Wn.shape[1]

  def body(s_ref, wn_ref, q_ref, d_ref, o_ref):
    invd = 1.0 / jnp.maximum(d_ref[...], 1.0)
    t = s_ref[...] + jnp.dot(q_ref[...] * invd, wn_ref[...], **_F32)
    if act:
      t = jnp.maximum(t, 0.0)
    o_ref[...] = t

  return pl.pallas_call(
      body,
      grid=(_G,),
      in_specs=[
          pl.BlockSpec((_BR, Dout), lambda i: (i, 0)),
          pl.BlockSpec((_D, Dout), lambda i: (0, 0)),
          pl.BlockSpec((_BR, _D), lambda i: (i, 0)),
          pl.BlockSpec((_BR, 1), lambda i: (i, 0)),
      ],
      out_specs=pl.BlockSpec((_BR, Dout), lambda i: (i, 0)),
      out_shape=jax.ShapeDtypeStruct((_N, Dout), jnp.float32),
  )(s, Wn, q, degc)


def kernel(edge_index, inputs, W0s, W0n, b0, W1s, W1n, b1, W2s, W2n, b2):
  ei = edge_index.astype(jnp.int32)
  src = ei[0].reshape(_ER, _CH)
  dst = ei[1].reshape(_ER, _CH)
  dstpf = jnp.pad(dst, ((0, 0), (0, 128 - _CH)),
                  constant_values=_N + 100).reshape(-1)

  b0r = b0.reshape(1, -1)
  b1r = b1.reshape(1, -1)
  b2r = b2.reshape(1, -1)

  scat = _make_scatter()
  deghist = _make_deghist()

  histo = deghist(dstpf)
  degc = _degsum(histo.reshape(_NS, _NP)).reshape(_NP, 1)
  s0 = _selfproj(inputs, W0s, b0r)
  q = scat(inputs, src, dst)
  h1 = _combine(s0, W0n, q, degc, False)
  s1 = _selfproj(h1, W1s, b1r)
  q = scat(h1, src, dst)
  h2 = _combine(s1, W1n, q, degc, True)
  s2 = _selfproj(h2, W1s, b1r)
  q = scat(h2, src, dst)
  h3 = _combine(s2, W1n, q, degc, True)
  s3 = _selfproj(h3, W2s, b2r)
  q = scat(h3, src, dst)
  out = _combine(s3, W2n, q, degc, False)
  return out, h3
